# initial kernel scaffold (unmeasured)
import jax
import jax.numpy as jnp
from jax import lax
from jax.experimental import pallas as pl
from jax.experimental.pallas import tpu as pltpu

N_DEV = 16
DH = 64


def kernel(x, Wq, Wo, K_ext, V_ext):
    B, S, D = x.shape
    H = Wq.shape[1] // DH
    my = lax.axis_index("i")

    k = lax.dynamic_slice_in_dim(K_ext, my * H, H, axis=2)
    k = jnp.transpose(k, (0, 2, 1, 3)).astype(jnp.bfloat16)
    v = lax.dynamic_slice_in_dim(V_ext, my * H, H, axis=2)
    v = jnp.transpose(v, (0, 2, 1, 3)).astype(jnp.bfloat16)

    xb = x.astype(jnp.bfloat16)
    wq = Wq.astype(jnp.bfloat16)
    wo = Wo.astype(jnp.bfloat16)

    def body(x_ref, wq_ref, wo_ref, k_ref, v_ref, out_ref,
             xbuf, abuf, xsend, xrecv, asend, arecv):
        my_i = lax.axis_index("i")
        right = lax.rem(my_i + 1, N_DEV)

        def partial_for(xblk):
            x2 = xblk.reshape(B * S, D)
            q = jnp.dot(x2, wq_ref[...], preferred_element_type=jnp.float32)
            q = q.astype(jnp.bfloat16).reshape(B, S, H, DH)
            s = lax.dot_general(
                q, k_ref[...],
                dimension_numbers=(((3,), (3,)), ((0, 2), (0, 1))),
                preferred_element_type=jnp.float32,
            ) * 0.125
            m = jnp.max(s, axis=-1, keepdims=True)
            p = jnp.exp(s - m)
            l = jnp.sum(p, axis=-1, keepdims=True)
            p = (p / l).astype(jnp.bfloat16)
            o = lax.dot_general(
                p, v_ref[...],
                dimension_numbers=(((3,), (2,)), ((0, 1), (0, 1))),
                preferred_element_type=jnp.float32,
            )
            o = jnp.transpose(o.astype(jnp.bfloat16), (0, 2, 1, 3))
            o = o.reshape(B * S, H * DH)
            r = jnp.dot(o, wo_ref[...], preferred_element_type=jnp.float32)
            return r.reshape(B, S, D)

        xbuf[0] = x_ref[...]
        out_ref[...] = partial_for(x_ref[...])

        for h in range(N_DEV):
            ops = []
            if h < N_DEV - 1:
                xr = pltpu.make_async_remote_copy(
                    src_ref=xbuf.at[h], dst_ref=xbuf.at[h + 1],
                    send_sem=xsend.at[h], recv_sem=xrecv.at[h],
                    device_id=(right,), device_id_type=pl.DeviceIdType.MESH,
                )
                xr.start()
                ops.append(xr)
            if h >= 1:
                dst_slot = h + 1 if h < N_DEV - 1 else 0
                ar = pltpu.make_async_remote_copy(
                    src_ref=abuf.at[h], dst_ref=abuf.at[dst_slot],
                    send_sem=asend.at[h], recv_sem=arecv.at[h],
                    device_id=(right,), device_id_type=pl.DeviceIdType.MESH,
                )
                ar.start()
                ops.append(ar)
            for op in ops:
                op.wait()
            if h < N_DEV - 1:
                part = partial_for(xbuf[h + 1])
                if h == 0:
                    abuf[1] = part
                else:
                    abuf[h + 1] = abuf[h + 1] + part

        out_ref[...] = out_ref[...] + abuf[0]

    return pl.pallas_call(
        body,
        out_shape=jax.ShapeDtypeStruct((B, S, D), jnp.float32),
        in_specs=[pl.BlockSpec(memory_space=pltpu.VMEM)] * 5,
        out_specs=pl.BlockSpec(memory_space=pltpu.VMEM),
        scratch_shapes=[
            pltpu.VMEM((N_DEV, B, S, D), jnp.bfloat16),
            pltpu.VMEM((N_DEV, B, S, D), jnp.float32),
            pltpu.SemaphoreType.DMA((N_DEV,)),
            pltpu.SemaphoreType.DMA((N_DEV,)),
            pltpu.SemaphoreType.DMA((N_DEV,)),
            pltpu.SemaphoreType.DMA((N_DEV,)),
        ],
        compiler_params=pltpu.CompilerParams(collective_id=0),
    )(xb, wq, wo, k, v)


# baseline (device time: 202486 ns/iter reference)
import jax
import jax.numpy as jnp
from jax import lax
from jax.experimental import pallas as pl
from jax.experimental.pallas import tpu as pltpu

N_DEV = 16
DH = 64


def kernel(x, Wq, Wo, K_ext, V_ext):
    B, S, D = x.shape
    H = Wq.shape[1] // DH
    my = lax.axis_index("i")

    Skv = K_ext.shape[1]
    k = lax.dynamic_slice_in_dim(K_ext, my * H, H, axis=2)
    k = jnp.transpose(k, (0, 2, 1, 3)).reshape(B * H, Skv, DH).astype(jnp.bfloat16)
    v = lax.dynamic_slice_in_dim(V_ext, my * H, H, axis=2)
    v = jnp.transpose(v, (0, 2, 1, 3)).reshape(B * H, Skv, DH).astype(jnp.bfloat16)

    xb = x.astype(jnp.bfloat16)
    wq = Wq.astype(jnp.bfloat16)
    wo = Wo.astype(jnp.bfloat16)

    def body(x_ref, wq_ref, wo_ref, k_ref, v_ref, out_ref,
             xbuf, abuf, xsend, xrecv, asend, arecv):
        my_i = lax.axis_index("i")
        right = lax.rem(my_i + 1, N_DEV)

        def partial_for(xblk):
            x2 = xblk.reshape(B * S, D)
            q = jnp.dot(x2, wq_ref[...], preferred_element_type=jnp.float32)
            q = q.astype(jnp.bfloat16).reshape(B, S, H, DH)
            q = jnp.transpose(q, (0, 2, 1, 3)).reshape(B * H, S, DH)
            s = lax.dot_general(
                q, k_ref[...],
                dimension_numbers=(((2,), (2,)), ((0,), (0,))),
                preferred_element_type=jnp.float32,
            ) * 0.125
            m = jnp.max(s, axis=-1, keepdims=True)
            p = jnp.exp(s - m)
            l = jnp.sum(p, axis=-1, keepdims=True)
            p = (p / l).astype(jnp.bfloat16)
            o = lax.dot_general(
                p, v_ref[...],
                dimension_numbers=(((2,), (1,)), ((0,), (0,))),
                preferred_element_type=jnp.float32,
            )
            o = o.astype(jnp.bfloat16).reshape(B, H, S, DH)
            o = jnp.transpose(o, (0, 2, 1, 3)).reshape(B * S, H * DH)
            r = jnp.dot(o, wo_ref[...], preferred_element_type=jnp.float32)
            return r.reshape(B, S, D)

        xbuf[0] = x_ref[...]
        out_ref[...] = partial_for(x_ref[...])

        for h in range(N_DEV):
            ops = []
            if h < N_DEV - 1:
                xr = pltpu.make_async_remote_copy(
                    src_ref=xbuf.at[h], dst_ref=xbuf.at[h + 1],
                    send_sem=xsend.at[h], recv_sem=xrecv.at[h],
                    device_id=(right,), device_id_type=pl.DeviceIdType.MESH,
                )
                xr.start()
                ops.append(xr)
            if h >= 1:
                dst_slot = h + 1 if h < N_DEV - 1 else 0
                ar = pltpu.make_async_remote_copy(
                    src_ref=abuf.at[h], dst_ref=abuf.at[dst_slot],
                    send_sem=asend.at[h], recv_sem=arecv.at[h],
                    device_id=(right,), device_id_type=pl.DeviceIdType.MESH,
                )
                ar.start()
                ops.append(ar)
            for op in ops:
                op.wait()
            if h < N_DEV - 1:
                part = partial_for(xbuf[h + 1])
                if h == 0:
                    abuf[1] = part
                else:
                    abuf[h + 1] = abuf[h + 1] + part

        out_ref[...] = out_ref[...] + abuf[0]

    return pl.pallas_call(
        body,
        out_shape=jax.ShapeDtypeStruct((B, S, D), jnp.float32),
        in_specs=[pl.BlockSpec(memory_space=pltpu.VMEM)] * 5,
        out_specs=pl.BlockSpec(memory_space=pltpu.VMEM),
        scratch_shapes=[
            pltpu.VMEM((N_DEV, B, S, D), jnp.bfloat16),
            pltpu.VMEM((N_DEV, B, S, D), jnp.float32),
            pltpu.SemaphoreType.DMA((N_DEV,)),
            pltpu.SemaphoreType.DMA((N_DEV,)),
            pltpu.SemaphoreType.DMA((N_DEV,)),
            pltpu.SemaphoreType.DMA((N_DEV,)),
        ],
    )(xb, wq, wo, k, v)


# device time: 160262 ns/iter; 1.2635x vs baseline; 1.2635x over previous
import jax
import jax.numpy as jnp
from jax import lax
from jax.experimental import pallas as pl
from jax.experimental.pallas import tpu as pltpu

N_DEV = 16
DH = 64


def kernel(x, Wq, Wo, K_ext, V_ext):
    B, S, D = x.shape
    H = Wq.shape[1] // DH
    my = lax.axis_index("i")

    Skv = K_ext.shape[1]
    k = lax.dynamic_slice_in_dim(K_ext, my * H, H, axis=2)
    k = jnp.transpose(k, (0, 2, 1, 3)).reshape(B * H, Skv, DH).astype(jnp.bfloat16)
    v = lax.dynamic_slice_in_dim(V_ext, my * H, H, axis=2)
    v = jnp.transpose(v, (0, 2, 1, 3)).reshape(B * H, Skv, DH).astype(jnp.bfloat16)

    xb = x.astype(jnp.bfloat16)
    wq = Wq.astype(jnp.bfloat16)
    wo = Wo.astype(jnp.bfloat16)

    def body(x_ref, wq_ref, wo_ref, k_ref, v_ref, out_ref,
             xbuf, abuf, xsend, xrecv, asend, arecv):
        my_i = lax.axis_index("i")
        right = lax.rem(my_i + 1, N_DEV)

        def partial_for(xblk):
            x2 = xblk.reshape(B * S, D)
            q = jnp.dot(x2, wq_ref[...], preferred_element_type=jnp.float32)
            q = q.astype(jnp.bfloat16).reshape(B, S, H, DH)
            q = jnp.transpose(q, (0, 2, 1, 3)).reshape(B * H, S, DH)
            s = lax.dot_general(
                q, k_ref[...],
                dimension_numbers=(((2,), (2,)), ((0,), (0,))),
                preferred_element_type=jnp.float32,
            ) * 0.125
            m = jnp.max(s, axis=-1, keepdims=True)
            p = jnp.exp(s - m)
            l = jnp.sum(p, axis=-1, keepdims=True)
            p = (p / l).astype(jnp.bfloat16)
            o = lax.dot_general(
                p, v_ref[...],
                dimension_numbers=(((2,), (1,)), ((0,), (0,))),
                preferred_element_type=jnp.float32,
            )
            o = o.astype(jnp.bfloat16).reshape(B, H, S, DH)
            o = jnp.transpose(o, (0, 2, 1, 3)).reshape(B * S, H * DH)
            r = jnp.dot(o, wo_ref[...], preferred_element_type=jnp.float32)
            return r.reshape(B, S, D)

        xbuf[0] = x_ref[...]
        out_ref[...] = partial_for(x_ref[...])

        for h in range(N_DEV):
            ops = []
            if h < N_DEV - 1:
                xr = pltpu.make_async_remote_copy(
                    src_ref=xbuf.at[h], dst_ref=xbuf.at[h + 1],
                    send_sem=xsend.at[h], recv_sem=xrecv.at[h],
                    device_id=(right,), device_id_type=pl.DeviceIdType.MESH,
                )
                xr.start()
                ops.append(xr)
            if h >= 1:
                dst_slot = h + 1 if h < N_DEV - 1 else 0
                ar = pltpu.make_async_remote_copy(
                    src_ref=abuf.at[h], dst_ref=abuf.at[dst_slot],
                    send_sem=asend.at[h], recv_sem=arecv.at[h],
                    device_id=(right,), device_id_type=pl.DeviceIdType.MESH,
                )
                ar.start()
                ops.append(ar)
            for op in ops:
                op.wait()
            if h < N_DEV - 1:
                part = partial_for(xbuf[h + 1])
                if h == 0:
                    abuf[1] = part.astype(jnp.bfloat16)
                else:
                    abuf[h + 1] = (
                        abuf[h + 1][...].astype(jnp.float32) + part
                    ).astype(jnp.bfloat16)

        out_ref[...] = out_ref[...] + abuf[0][...].astype(jnp.float32)

    return pl.pallas_call(
        body,
        out_shape=jax.ShapeDtypeStruct((B, S, D), jnp.float32),
        in_specs=[pl.BlockSpec(memory_space=pltpu.VMEM)] * 5,
        out_specs=pl.BlockSpec(memory_space=pltpu.VMEM),
        scratch_shapes=[
            pltpu.VMEM((N_DEV, B, S, D), jnp.bfloat16),
            pltpu.VMEM((N_DEV, B, S, D), jnp.bfloat16),
            pltpu.SemaphoreType.DMA((N_DEV,)),
            pltpu.SemaphoreType.DMA((N_DEV,)),
            pltpu.SemaphoreType.DMA((N_DEV,)),
            pltpu.SemaphoreType.DMA((N_DEV,)),
        ],
    )(xb, wq, wo, k, v)


# device time: 96444 ns/iter; 2.0995x vs baseline; 1.6617x over previous
import jax
import jax.numpy as jnp
from jax import lax
from jax.experimental import pallas as pl
from jax.experimental.pallas import tpu as pltpu

N_DEV = 16
DH = 64


def kernel(x, Wq, Wo, K_ext, V_ext):
    B, S, D = x.shape
    H = Wq.shape[1] // DH
    my = lax.axis_index("i")

    Skv = K_ext.shape[1]
    k = lax.dynamic_slice_in_dim(K_ext, my * H, H, axis=2)
    k = jnp.transpose(k, (0, 2, 1, 3)).reshape(B * H, Skv, DH).astype(jnp.bfloat16)
    v = lax.dynamic_slice_in_dim(V_ext, my * H, H, axis=2)
    v = jnp.transpose(v, (0, 2, 1, 3)).reshape(B * H, Skv, DH).astype(jnp.bfloat16)

    xb = x.astype(jnp.bfloat16)
    wq = Wq.astype(jnp.bfloat16)
    wo = Wo.astype(jnp.bfloat16)

    def body(x_ref, wq_ref, wo_ref, k_ref, v_ref, out_ref,
             xbuf, abuf, xsend, xrecv, asend, arecv):
        my_i = lax.axis_index("i")
        right = lax.rem(my_i + 1, N_DEV)

        def partial_for(xblk):
            x2 = xblk.reshape(B * S, D)
            q = jnp.dot(x2, wq_ref[...], preferred_element_type=jnp.float32)
            q = q.astype(jnp.bfloat16).reshape(B, S, H, DH)
            q = jnp.transpose(q, (0, 2, 1, 3)).reshape(B * H, S, DH)
            s = lax.dot_general(
                q, k_ref[...],
                dimension_numbers=(((2,), (2,)), ((0,), (0,))),
                preferred_element_type=jnp.float32,
            ) * 0.125
            m = jnp.max(s, axis=-1, keepdims=True)
            p = jnp.exp(s - m)
            l = jnp.sum(p, axis=-1, keepdims=True)
            p = (p / l).astype(jnp.bfloat16)
            o = lax.dot_general(
                p, v_ref[...],
                dimension_numbers=(((2,), (1,)), ((0,), (0,))),
                preferred_element_type=jnp.float32,
            )
            o = o.astype(jnp.bfloat16).reshape(B, H, S, DH)
            o = jnp.transpose(o, (0, 2, 1, 3)).reshape(B * S, H * DH)
            r = jnp.dot(o, wo_ref[...], preferred_element_type=jnp.float32)
            return r.reshape(B, S, D)

        left = lax.rem(my_i + N_DEV - 1, N_DEV)

        def make_x(h):
            xr = pltpu.make_async_remote_copy(
                src_ref=xbuf.at[h, 0], dst_ref=xbuf.at[h + 1, 0],
                send_sem=xsend.at[h, 0], recv_sem=xrecv.at[h, 0],
                device_id=(right,), device_id_type=pl.DeviceIdType.MESH,
            )
            xl = pltpu.make_async_remote_copy(
                src_ref=xbuf.at[h, 1], dst_ref=xbuf.at[h + 1, 1],
                send_sem=xsend.at[h, 1], recv_sem=xrecv.at[h, 1],
                device_id=(left,), device_id_type=pl.DeviceIdType.MESH,
            )
            return xr, xl

        def make_a(h):
            dst = h + 1 if h < N_DEV - 1 else 0
            ar = pltpu.make_async_remote_copy(
                src_ref=abuf.at[h, 0], dst_ref=abuf.at[dst, 0],
                send_sem=asend.at[h, 0], recv_sem=arecv.at[h, 0],
                device_id=(right,), device_id_type=pl.DeviceIdType.MESH,
            )
            al = pltpu.make_async_remote_copy(
                src_ref=abuf.at[h, 1], dst_ref=abuf.at[dst, 1],
                send_sem=asend.at[h, 1], recv_sem=arecv.at[h, 1],
                device_id=(left,), device_id_type=pl.DeviceIdType.MESH,
            )
            return ar, al

        xbuf[0] = x_ref[...]
        xd = make_x(0)
        xd[0].start()
        xd[1].start()
        out_ref[...] = partial_for(x_ref[...])

        for h in range(N_DEV):
            if h >= 1:
                ad = make_a(h)
                ad[0].start()
                ad[1].start()
            if h <= N_DEV - 2:
                xd[0].wait()
                xd[1].wait()
                if h <= N_DEV - 3:
                    xd = make_x(h + 1)
                    xd[0].start()
                    xd[1].start()
                part = partial_for(xbuf[h + 1])
            if h >= 1:
                ad[0].wait()
                ad[1].wait()
            if h == 0:
                abuf[1] = part.astype(jnp.bfloat16)
            elif h <= N_DEV - 2:
                abuf[h + 1] = (
                    abuf[h + 1].astype(jnp.float32) + part
                ).astype(jnp.bfloat16)

        out_ref[...] = out_ref[...] + abuf[0].astype(jnp.float32)

    return pl.pallas_call(
        body,
        out_shape=jax.ShapeDtypeStruct((B, S, D), jnp.float32),
        in_specs=[pl.BlockSpec(memory_space=pltpu.VMEM)] * 5,
        out_specs=pl.BlockSpec(memory_space=pltpu.VMEM),
        scratch_shapes=[
            pltpu.VMEM((N_DEV, B, S, D), jnp.bfloat16),
            pltpu.VMEM((N_DEV, B, S, D), jnp.bfloat16),
            pltpu.SemaphoreType.DMA((N_DEV, 2)),
            pltpu.SemaphoreType.DMA((N_DEV, 2)),
            pltpu.SemaphoreType.DMA((N_DEV, 2)),
            pltpu.SemaphoreType.DMA((N_DEV, 2)),
        ],
    )(xb, wq, wo, k, v)


# device time: 73105 ns/iter; 2.7698x vs baseline; 1.3193x over previous
import jax
import jax.numpy as jnp
from jax import lax
from jax.experimental import pallas as pl
from jax.experimental.pallas import tpu as pltpu

N_DEV = 16
DH = 64

_NEXT = [4, 2, 6, 0, 8, 1, 10, 3, 12, 5, 14, 7, 13, 9, 15, 11]
_PREV = [3, 5, 1, 7, 0, 9, 2, 11, 4, 13, 6, 15, 8, 12, 10, 14]


def kernel(x, Wq, Wo, K_ext, V_ext):
    B, S, D = x.shape
    H = Wq.shape[1] // DH
    my = lax.axis_index("i")

    Skv = K_ext.shape[1]
    k = lax.dynamic_slice_in_dim(K_ext, my * H, H, axis=2)
    k = jnp.transpose(k, (0, 2, 1, 3)).reshape(B * H, Skv, DH).astype(jnp.bfloat16)
    v = lax.dynamic_slice_in_dim(V_ext, my * H, H, axis=2)
    v = jnp.transpose(v, (0, 2, 1, 3)).reshape(B * H, Skv, DH).astype(jnp.bfloat16)

    xb = x.astype(jnp.bfloat16)
    wq = Wq.astype(jnp.bfloat16)
    wo = Wo.astype(jnp.bfloat16)

    nxt = jnp.asarray(_NEXT, dtype=jnp.int32)[my].reshape(1)
    prv = jnp.asarray(_PREV, dtype=jnp.int32)[my].reshape(1)

    def body(x_ref, wq_ref, wo_ref, k_ref, v_ref, nxt_ref, prv_ref, out_ref,
             xbuf, abuf, xsend, xrecv, asend, arecv):
        right = nxt_ref[0]

        def partial_for(xblk):
            x2 = xblk.reshape(B * S, D)
            q = jnp.dot(x2, wq_ref[...], preferred_element_type=jnp.float32)
            q = q.astype(jnp.bfloat16).reshape(B, S, H, DH)
            q = jnp.transpose(q, (0, 2, 1, 3)).reshape(B * H, S, DH)
            s = lax.dot_general(
                q, k_ref[...],
                dimension_numbers=(((2,), (2,)), ((0,), (0,))),
                preferred_element_type=jnp.float32,
            ) * 0.125
            m = jnp.max(s, axis=-1, keepdims=True)
            p = jnp.exp(s - m)
            l = jnp.sum(p, axis=-1, keepdims=True)
            p = (p / l).astype(jnp.bfloat16)
            o = lax.dot_general(
                p, v_ref[...],
                dimension_numbers=(((2,), (1,)), ((0,), (0,))),
                preferred_element_type=jnp.float32,
            )
            o = o.astype(jnp.bfloat16).reshape(B, H, S, DH)
            o = jnp.transpose(o, (0, 2, 1, 3)).reshape(B * S, H * DH)
            r = jnp.dot(o, wo_ref[...], preferred_element_type=jnp.float32)
            return r.reshape(B, S, D)

        left = prv_ref[0]

        barrier = pltpu.get_barrier_semaphore()
        pl.semaphore_signal(barrier, inc=1, device_id=(left,),
                            device_id_type=pl.DeviceIdType.MESH)
        pl.semaphore_signal(barrier, inc=1, device_id=(right,),
                            device_id_type=pl.DeviceIdType.MESH)
        pl.semaphore_wait(barrier, 2)

        def make_x(h):
            xr = pltpu.make_async_remote_copy(
                src_ref=xbuf.at[h, 0], dst_ref=xbuf.at[h + 1, 0],
                send_sem=xsend.at[h, 0], recv_sem=xrecv.at[h, 0],
                device_id=(right,), device_id_type=pl.DeviceIdType.MESH,
            )
            xl = pltpu.make_async_remote_copy(
                src_ref=xbuf.at[h, 1], dst_ref=xbuf.at[h + 1, 1],
                send_sem=xsend.at[h, 1], recv_sem=xrecv.at[h, 1],
                device_id=(left,), device_id_type=pl.DeviceIdType.MESH,
            )
            return xr, xl

        def make_a(h):
            dst = h + 1 if h < N_DEV - 1 else 0
            ar = pltpu.make_async_remote_copy(
                src_ref=abuf.at[h, 0], dst_ref=abuf.at[dst, 0],
                send_sem=asend.at[h, 0], recv_sem=arecv.at[h, 0],
                device_id=(right,), device_id_type=pl.DeviceIdType.MESH,
            )
            al = pltpu.make_async_remote_copy(
                src_ref=abuf.at[h, 1], dst_ref=abuf.at[dst, 1],
                send_sem=asend.at[h, 1], recv_sem=arecv.at[h, 1],
                device_id=(left,), device_id_type=pl.DeviceIdType.MESH,
            )
            return ar, al

        xbuf[0] = x_ref[...]
        xd = make_x(0)
        xd[0].start()
        xd[1].start()
        out_ref[...] = partial_for(x_ref[...])

        for h in range(N_DEV):
            if h >= 1:
                ad = make_a(h)
                ad[0].start()
                ad[1].start()
            if h <= N_DEV - 2:
                xd[0].wait()
                xd[1].wait()
                if h <= N_DEV - 3:
                    xd = make_x(h + 1)
                    xd[0].start()
                    xd[1].start()
                part = partial_for(xbuf[h + 1])
            if h >= 1:
                ad[0].wait()
                ad[1].wait()
            if h == 0:
                abuf[1] = part.astype(jnp.bfloat16)
            elif h <= N_DEV - 2:
                abuf[h + 1] = (
                    abuf[h + 1].astype(jnp.float32) + part
                ).astype(jnp.bfloat16)

        out_ref[...] = out_ref[...] + abuf[0].astype(jnp.float32)

    return pl.pallas_call(
        body,
        out_shape=jax.ShapeDtypeStruct((B, S, D), jnp.float32),
        in_specs=[pl.BlockSpec(memory_space=pltpu.VMEM)] * 5
        + [pl.BlockSpec(memory_space=pltpu.SMEM)] * 2,
        out_specs=pl.BlockSpec(memory_space=pltpu.VMEM),
        scratch_shapes=[
            pltpu.VMEM((N_DEV, B, S, D), jnp.bfloat16),
            pltpu.VMEM((N_DEV, B, S, D), jnp.bfloat16),
            pltpu.SemaphoreType.DMA((N_DEV, 2)),
            pltpu.SemaphoreType.DMA((N_DEV, 2)),
            pltpu.SemaphoreType.DMA((N_DEV, 2)),
            pltpu.SemaphoreType.DMA((N_DEV, 2)),
        ],
        compiler_params=pltpu.CompilerParams(collective_id=0),
    )(xb, wq, wo, k, v, nxt, prv)
